# Initial kernel scaffold; baseline (speedup 1.0000x reference)
import jax
import jax.numpy as jnp
from jax.experimental import pallas as pl


def kernel(x, Wr, br, gr, betar, W1, b1, g1, bt1, W2, b2, g2, bt2):
    raise NotImplementedError("write your pallas kernel here")


# speedup vs baseline: 1.4208x; 1.4208x over previous
import functools

import jax
import jax.numpy as jnp
from jax.experimental import pallas as pl
from jax.experimental.pallas import tpu as pltpu

T, B, C, H, W = 4, 16, 256, 14, 14
E, K = 8, 2
HID, OUT = 256, 256
HW = H * W
TB = T * B
_C1 = 1.0 / (1.0 + 1e-5) ** 0.5


def _router_body(x_ref, wr_ref, shift_ref, out_ref):
    v = jnp.zeros((C, HW), jnp.float32)
    ms = []
    for t in range(T):
        v = (v + x_ref[t, 0]) * 0.5
        s = (v >= 1.0).astype(jnp.float32)
        v = v * (1.0 - s)
        ms.append(jnp.sum(s, axis=-1))
    m = jnp.stack(ms, axis=0) * (1.0 / HW)
    out_ref[0] = (
        jnp.dot(m, wr_ref[...], preferred_element_type=jnp.float32) + shift_ref[...]
    )


def _expert_body(idx_ref, tau_ref, wk_ref, x_ref, w1_ref, w2_ref, d1_ref, d2_ref,
                 out_ref):
    k = pl.program_id(1)
    p = pl.program_id(0) * K + k
    tau = tau_ref[p]
    wgt = wk_ref[p]
    x = x_ref[0]
    s1 = (x >= tau).astype(jnp.float32)
    h = jnp.dot(w1_ref[0], s1, preferred_element_type=jnp.float32) + d1_ref[0, 0][:, None]
    x2 = x + h
    s2 = (x2 >= tau).astype(jnp.float32)
    o = jnp.dot(w2_ref[0], s2, preferred_element_type=jnp.float32) + d2_ref[0, 0][:, None]
    res = (o + x2) * wgt

    @pl.when(k == 0)
    def _():
        out_ref[0] = res

    @pl.when(k == 1)
    def _():
        out_ref[0] = out_ref[0] + res


def kernel(x, Wr, br, gr, betar, W1, b1, g1, bt1, W2, b2, g2, bt2):
    f32 = jnp.float32
    x4 = x.reshape(T, B, C, HW)

    wr_s = Wr.T * (gr * _C1)[None, :]
    shift = (br * gr * _C1 + betar)[None, :]
    logits_bt = pl.pallas_call(
        _router_body,
        grid=(B,),
        in_specs=[
            pl.BlockSpec((T, 1, C, HW), lambda b: (0, b, 0, 0)),
            pl.BlockSpec((C, E), lambda b: (0, 0)),
            pl.BlockSpec((1, E), lambda b: (0, 0)),
        ],
        out_specs=pl.BlockSpec((1, T, E), lambda b: (b, 0, 0)),
        out_shape=jax.ShapeDtypeStruct((B, T, E), f32),
    )(x4, wr_s, shift)
    logits = logits_bt.transpose(1, 0, 2).reshape(TB, E)

    probs = jax.nn.softmax(logits, axis=-1)
    wk, idx = jax.lax.top_k(probs, K)
    wk = wk / jnp.sum(wk, axis=-1, keepdims=True)

    taus = jnp.linspace(1.5, 4.0, E).astype(f32)
    idx_p = idx.reshape(-1).astype(jnp.int32)
    tau_p = taus[idx_p]
    wk_p = wk.reshape(-1).astype(f32)

    w1g = W1 * (g1 * _C1)[:, :, None]
    w2g = W2 * (g2 * _C1)[:, :, None]
    d1 = (b1 * g1 * _C1 + bt1).reshape(E, 1, HID)
    d2 = (b2 * g2 * _C1 + bt2).reshape(E, 1, OUT)
    xt = x4.reshape(TB, C, HW)

    out = pl.pallas_call(
        _expert_body,
        grid_spec=pltpu.PrefetchScalarGridSpec(
            num_scalar_prefetch=3,
            grid=(TB, K),
            in_specs=[
                pl.BlockSpec((1, C, HW), lambda t, k, i, ta, wv: (t, 0, 0)),
                pl.BlockSpec((1, HID, C), lambda t, k, i, ta, wv: (i[t * K + k], 0, 0)),
                pl.BlockSpec((1, OUT, HID), lambda t, k, i, ta, wv: (i[t * K + k], 0, 0)),
                pl.BlockSpec((1, 1, HID), lambda t, k, i, ta, wv: (i[t * K + k], 0, 0)),
                pl.BlockSpec((1, 1, OUT), lambda t, k, i, ta, wv: (i[t * K + k], 0, 0)),
            ],
            out_specs=pl.BlockSpec((1, OUT, HW), lambda t, k, i, ta, wv: (t, 0, 0)),
        ),
        out_shape=jax.ShapeDtypeStruct((TB, OUT, HW), f32),
    )(idx_p, tau_p, wk_p, xt, w1g, w2g, d1, d2)

    return out.reshape(T, B, OUT, H, W)


# speedup vs baseline: 2.4211x; 1.7040x over previous
import functools

import jax
import jax.numpy as jnp
from jax.experimental import pallas as pl
from jax.experimental.pallas import tpu as pltpu

T, B, C, H, W = 4, 16, 256, 14, 14
E, K = 8, 2
HID, OUT = 256, 256
HW = H * W
TB = T * B
_C1 = 1.0 / (1.0 + 1e-5) ** 0.5


def _router_body(x_ref, wr_ref, shift_ref, out_ref):
    v = jnp.zeros((C, HW), jnp.float32)
    ms = []
    for t in range(T):
        v = (v + x_ref[t, 0]) * 0.5
        s = (v >= 1.0).astype(jnp.float32)
        v = v * (1.0 - s)
        ms.append(jnp.sum(s, axis=-1))
    m = jnp.stack(ms, axis=0) * (1.0 / HW)
    out_ref[0] = (
        jnp.dot(m, wr_ref[...], preferred_element_type=jnp.float32) + shift_ref[...]
    )


CHUNK = 8


def _expert_body(idx_ref, wk_ref, x_ref, w1_ref, w2_ref, d1_ref, d2_ref, out_ref):
    n = pl.program_id(0)
    tau_step = jnp.float32(2.5 / (E - 1))
    for j in range(CHUNK):
        t = n * CHUNK + j
        x = x_ref[j]
        acc = None
        for k in range(K):
            e = idx_ref[t * K + k]
            tau = 1.5 + e.astype(jnp.float32) * tau_step
            wgt = wk_ref[t * K + k]
            s1 = (x >= tau).astype(jnp.float32)
            h = (jnp.dot(w1_ref[e], s1, preferred_element_type=jnp.float32)
                 + d1_ref[e, 0][:, None])
            x2 = x + h
            s2 = (x2 >= tau).astype(jnp.float32)
            o = (jnp.dot(w2_ref[e], s2, preferred_element_type=jnp.float32)
                 + d2_ref[e, 0][:, None])
            res = (o + x2) * wgt
            acc = res if k == 0 else acc + res
        out_ref[j] = acc


def kernel(x, Wr, br, gr, betar, W1, b1, g1, bt1, W2, b2, g2, bt2):
    f32 = jnp.float32
    x4 = x.reshape(T, B, C, HW)

    wr_s = Wr.T * (gr * _C1)[None, :]
    shift = (br * gr * _C1 + betar)[None, :]
    logits_bt = pl.pallas_call(
        _router_body,
        grid=(B,),
        in_specs=[
            pl.BlockSpec((T, 1, C, HW), lambda b: (0, b, 0, 0)),
            pl.BlockSpec((C, E), lambda b: (0, 0)),
            pl.BlockSpec((1, E), lambda b: (0, 0)),
        ],
        out_specs=pl.BlockSpec((1, T, E), lambda b: (b, 0, 0)),
        out_shape=jax.ShapeDtypeStruct((B, T, E), f32),
    )(x4, wr_s, shift)
    logits = logits_bt.transpose(1, 0, 2).reshape(TB, E)

    probs = jax.nn.softmax(logits, axis=-1)
    wk, idx = jax.lax.top_k(probs, K)
    wk = wk / jnp.sum(wk, axis=-1, keepdims=True)

    idx_p = idx.reshape(-1).astype(jnp.int32)
    wk_p = wk.reshape(-1).astype(f32)

    w1g = W1 * (g1 * _C1)[:, :, None]
    w2g = W2 * (g2 * _C1)[:, :, None]
    d1 = (b1 * g1 * _C1 + bt1).reshape(E, 1, HID)
    d2 = (b2 * g2 * _C1 + bt2).reshape(E, 1, OUT)
    xt = x4.reshape(TB, C, HW)

    out = pl.pallas_call(
        _expert_body,
        grid_spec=pltpu.PrefetchScalarGridSpec(
            num_scalar_prefetch=2,
            grid=(TB // CHUNK,),
            in_specs=[
                pl.BlockSpec((CHUNK, C, HW), lambda n, i, wv: (n, 0, 0)),
                pl.BlockSpec((E, HID, C), lambda n, i, wv: (0, 0, 0)),
                pl.BlockSpec((E, OUT, HID), lambda n, i, wv: (0, 0, 0)),
                pl.BlockSpec((E, 1, HID), lambda n, i, wv: (0, 0, 0)),
                pl.BlockSpec((E, 1, OUT), lambda n, i, wv: (0, 0, 0)),
            ],
            out_specs=pl.BlockSpec((CHUNK, OUT, HW), lambda n, i, wv: (n, 0, 0)),
        ),
        out_shape=jax.ShapeDtypeStruct((TB, OUT, HW), f32),
    )(idx_p, wk_p, xt, w1g, w2g, d1, d2)

    return out.reshape(T, B, OUT, H, W)
